# Initial kernel scaffold; baseline (speedup 1.0000x reference)
import jax
import jax.numpy as jnp
from jax.experimental import pallas as pl


def kernel(x, W1, W2, W3, W4, W5, L1, L2, b2):
    raise NotImplementedError("write your pallas kernel here")


# speedup vs baseline: 15.9227x; 15.9227x over previous
import functools

import jax
import jax.numpy as jnp
from jax import lax
from jax.experimental import pallas as pl
from jax.experimental.pallas import tpu as pltpu
from jax.experimental.pallas import tpu_sc as plsc

B = 16
N = 2048
P = B * N
K = 10
R = 256
NEG = -1e30


def _lrelu(v):
    return jnp.maximum(v, 0.2 * v)


def _knn_body(xr_ref, xt_ref, idx_ref):
    b = pl.program_id(0)
    r = pl.program_id(1)
    xr = xr_ref[0]
    xt = xt_ref[0]

    g = lax.dot_general(xr, xt, (((1,), (1,)), ((), ())),
                        preferred_element_type=jnp.float32)
    nr = jnp.sum(xr * xr, axis=1, keepdims=True)
    nc = jnp.sum(xt * xt, axis=1, keepdims=True)
    pw = (2.0 * g - nr) - nc.T

    col = lax.broadcasted_iota(jnp.int32, (R, N), 1)
    row_ids = r * R + lax.broadcasted_iota(jnp.int32, (R, 1), 0)
    pw = jnp.where(col == row_ids, NEG, pw)

    lane = lax.broadcasted_iota(jnp.int32, (R, 128), 1)
    idx_acc = jnp.where(lane == 0, row_ids, 0)
    for t in range(1, K):
        m = jnp.max(pw, axis=1, keepdims=True)
        cand = jnp.where(pw == m, col, N)
        a = jnp.min(cand, axis=1, keepdims=True)
        idx_acc = jnp.where(lane == t, a, idx_acc)
        pw = jnp.where(col == a, NEG, pw)

    idx_ref[0] = idx_acc[:, :K] + b * N


def _knn(xt):
    C = xt.shape[-1]
    return pl.pallas_call(
        _knn_body,
        grid=(B, N // R),
        in_specs=[
            pl.BlockSpec((1, R, C), lambda b, r: (b, r, 0)),
            pl.BlockSpec((1, N, C), lambda b, r: (b, 0, 0)),
        ],
        out_specs=pl.BlockSpec((1, R, K), lambda b, r: (b, r, 0)),
        out_shape=jax.ShapeDtypeStruct((B, N, K), jnp.int32),
    )(xt, xt)


def _gather_rows(x, idxk, C, SR):
    TOT = K * P
    NW = 32
    chunk = TOT // NW
    nsub = chunk // SR
    mesh = plsc.VectorSubcoreMesh(core_axis_name="c", subcore_axis_name="s")

    @functools.partial(
        pl.kernel,
        out_type=jax.ShapeDtypeStruct((TOT, C), jnp.float32),
        mesh=mesh,
        scratch_types=[
            pltpu.VMEM((SR,), jnp.int32),
            pltpu.VMEM((SR, C), jnp.float32),
            pltpu.SemaphoreType.DMA,
        ],
        compiler_params=pltpu.CompilerParams(use_tc_tiling_on_sc=False),
    )
    def sc_kernel(x_hbm, idx_hbm, out_hbm, idx_v, rows_v, sem):
        wid = lax.axis_index("s") * 2 + lax.axis_index("c")
        base = wid * chunk

        def sub(i, carry):
            r0 = base + i * SR
            pltpu.sync_copy(idx_hbm.at[pl.ds(r0, SR)], idx_v)
            pltpu.async_copy(x_hbm.at[idx_v], rows_v, sem).wait()
            pltpu.sync_copy(rows_v, out_hbm.at[pl.ds(r0, SR)])
            return carry

        lax.fori_loop(0, nsub, sub, 0)

    return sc_kernel(x, idxk)


def _conv_max_body(x_ref, xg_ref, w_ref, out_ref):
    xi = x_ref[...]
    w = w_ref[...]
    acc = None
    for j in range(K):
        xj = xg_ref[j]
        f2 = jnp.concatenate([xj - xi, xi], axis=1)
        h = lax.dot_general(f2, w, (((1,), (1,)), ((), ())),
                            preferred_element_type=jnp.float32)
        acc = h if acc is None else jnp.maximum(acc, h)
    out_ref[...] = _lrelu(acc)


def _conv_max(x, xg, w, O):
    C = x.shape[-1]
    xg = xg.reshape(K, P, C)
    return pl.pallas_call(
        _conv_max_body,
        grid=(P // R,),
        in_specs=[
            pl.BlockSpec((R, C), lambda t: (t, 0)),
            pl.BlockSpec((K, R, C), lambda t: (0, t, 0)),
            pl.BlockSpec((O, 2 * C), lambda t: (0, 0)),
        ],
        out_specs=pl.BlockSpec((R, O), lambda t: (t, 0)),
        out_shape=jax.ShapeDtypeStruct((P, O), jnp.float32),
    )(x, xg, w)


def _head_body(x1_ref, x2_ref, x3_ref, x4_ref, w1_ref, w2_ref, w3_ref,
               w4_ref, l1_ref, l2_ref, b2_ref, out_ref):
    dims = (((1,), (1,)), ((), ()))
    h = lax.dot_general(x1_ref[...], w1_ref[...], dims,
                        preferred_element_type=jnp.float32)
    h = h + lax.dot_general(x2_ref[...], w2_ref[...], dims,
                            preferred_element_type=jnp.float32)
    h = h + lax.dot_general(x3_ref[...], w3_ref[...], dims,
                            preferred_element_type=jnp.float32)
    h = h + lax.dot_general(x4_ref[...], w4_ref[...], dims,
                            preferred_element_type=jnp.float32)
    h = _lrelu(h)
    p1 = jnp.max(h, axis=0, keepdims=True)
    p2 = jnp.sum(h, axis=0, keepdims=True) * (1.0 / N)
    g = jnp.concatenate([p1, p2], axis=1)
    g = _lrelu(lax.dot_general(g, l1_ref[...], dims,
                               preferred_element_type=jnp.float32))
    g = _lrelu(lax.dot_general(g, l2_ref[...], dims,
                               preferred_element_type=jnp.float32)
               + b2_ref[...])
    out_ref[0] = g


def _head(x1, x2, x3, x4, w51, w52, w53, w54, L1, L2, b2):
    full = lambda shape: pl.BlockSpec(shape, lambda b: tuple(0 for _ in shape))
    return pl.pallas_call(
        _head_body,
        grid=(B,),
        in_specs=[
            pl.BlockSpec((N, 64), lambda b: (b, 0)),
            pl.BlockSpec((N, 64), lambda b: (b, 0)),
            pl.BlockSpec((N, 128), lambda b: (b, 0)),
            pl.BlockSpec((N, 256), lambda b: (b, 0)),
            full((512, 64)),
            full((512, 64)),
            full((512, 128)),
            full((512, 256)),
            full((512, 1024)),
            full((256, 512)),
            full((1, 256)),
        ],
        out_specs=pl.BlockSpec((1, 1, 256), lambda b: (b, 0, 0)),
        out_shape=jax.ShapeDtypeStruct((B, 1, 256), jnp.float32),
    )(x1, x2, x3, x4, w51, w52, w53, w54, L1, L2, b2).reshape(B, 256)


def _edge_block(xt, W, O, SR):
    C = xt.shape[-1]
    idx = _knn(xt)
    idxk = jnp.transpose(idx.reshape(P, K)).reshape(K * P)
    xf = xt.reshape(P, C)
    xg = _gather_rows(xf, idxk, C, SR)
    return _conv_max(xf, xg, W, O).reshape(B, N, O)


def kernel(x, W1, W2, W3, W4, W5, L1, L2, b2):
    xt = jnp.transpose(x, (0, 2, 1))
    xt = jnp.pad(xt, ((0, 0), (0, 0), (0, 13)))
    W1p = jnp.pad(W1.reshape(64, 2, 3),
                  ((0, 0), (0, 0), (0, 13))).reshape(64, 32)

    x1 = _edge_block(xt, W1p, 64, 1024)
    x2 = _edge_block(x1, W2, 64, 1024)
    x3 = _edge_block(x2, W3, 128, 512)
    x4 = _edge_block(x3, W4, 256, 512)

    return _head(x1.reshape(P, 64), x2.reshape(P, 64),
                 x3.reshape(P, 128), x4.reshape(P, 256),
                 W5[:, 0:64], W5[:, 64:128], W5[:, 128:256], W5[:, 256:512],
                 L1, L2, b2.reshape(1, 256))


# speedup vs baseline: 20.4963x; 1.2872x over previous
import functools

import jax
import jax.numpy as jnp
from jax import lax
from jax.experimental import pallas as pl
from jax.experimental.pallas import tpu as pltpu
from jax.experimental.pallas import tpu_sc as plsc

B = 16
N = 2048
P = B * N
K = 10
R = 512
NEG = -1e30


def _lrelu(v):
    return jnp.maximum(v, 0.2 * v)


def _knn_body(xr_ref, xt_ref, idx_ref, nc_ref):
    b = pl.program_id(0)
    r = pl.program_id(1)
    xr = xr_ref[0]
    xt = xt_ref[0]

    @pl.when(r == 0)
    def _():
        nc_ref[...] = jnp.sum(xt * xt, axis=1, keepdims=True).T

    g = lax.dot_general(xr, xt, (((1,), (1,)), ((), ())),
                        preferred_element_type=jnp.float32)
    nr = jnp.sum(xr * xr, axis=1, keepdims=True)
    pw = (2.0 * g - nr) - nc_ref[...]

    col = lax.broadcasted_iota(jnp.int32, (R, N), 1)
    row_ids = r * R + lax.broadcasted_iota(jnp.int32, (R, 1), 0)
    pw = jnp.where(col == row_ids, NEG, pw)

    lane = lax.broadcasted_iota(jnp.int32, (R, 128), 1)
    idx_acc = jnp.where(lane == 0, row_ids, 0)
    a = None
    for t in range(1, K):
        if a is not None:
            pw = jnp.where(col == a, NEG, pw)
        val = pw[:, 0:128]
        vidx = lane
        for c in range(1, N // 128):
            ch = pw[:, c * 128:(c + 1) * 128]
            gt = ch > val
            val = jnp.maximum(val, ch)
            vidx = jnp.where(gt, lane + c * 128, vidx)
        m = jnp.max(val, axis=1, keepdims=True)
        cand = jnp.where(val == m, vidx, N)
        a = jnp.min(cand, axis=1, keepdims=True)
        idx_acc = jnp.where(lane == t, a, idx_acc)

    idx_ref[0] = idx_acc[:, :K] + b * N


def _knn(xt):
    C = xt.shape[-1]
    return pl.pallas_call(
        _knn_body,
        grid=(B, N // R),
        in_specs=[
            pl.BlockSpec((1, R, C), lambda b, r: (b, r, 0)),
            pl.BlockSpec((1, N, C), lambda b, r: (b, 0, 0)),
        ],
        out_specs=pl.BlockSpec((1, R, K), lambda b, r: (b, r, 0)),
        out_shape=jax.ShapeDtypeStruct((B, N, K), jnp.int32),
        scratch_shapes=[pltpu.VMEM((1, N), jnp.float32)],
    )(xt, xt)


def _gather_rows(x, idxk, C, SR):
    TOT = K * P
    NW = 32
    chunk = TOT // NW
    nsub = chunk // SR
    mesh = plsc.VectorSubcoreMesh(core_axis_name="c", subcore_axis_name="s")

    @functools.partial(
        pl.kernel,
        out_type=jax.ShapeDtypeStruct((TOT, C), jnp.float32),
        mesh=mesh,
        scratch_types=[
            pltpu.VMEM((SR,), jnp.int32),
            pltpu.VMEM((SR, C), jnp.float32),
            pltpu.SemaphoreType.DMA,
        ],
        compiler_params=pltpu.CompilerParams(use_tc_tiling_on_sc=False),
    )
    def sc_kernel(x_hbm, idx_hbm, out_hbm, idx_v, rows_v, sem):
        wid = lax.axis_index("s") * 2 + lax.axis_index("c")
        base = wid * chunk

        def sub(i, carry):
            r0 = base + i * SR
            pltpu.sync_copy(idx_hbm.at[pl.ds(r0, SR)], idx_v)
            pltpu.async_copy(x_hbm.at[idx_v], rows_v, sem).wait()
            pltpu.sync_copy(rows_v, out_hbm.at[pl.ds(r0, SR)])
            return carry

        lax.fori_loop(0, nsub, sub, 0)

    return sc_kernel(x, idxk)


def _conv_max_body(x_ref, xg_ref, w_ref, out_ref):
    xi = x_ref[...]
    w = w_ref[...]
    acc = None
    for j in range(K):
        xj = xg_ref[j]
        f2 = jnp.concatenate([xj - xi, xi], axis=1)
        h = lax.dot_general(f2, w, (((1,), (1,)), ((), ())),
                            preferred_element_type=jnp.float32)
        acc = h if acc is None else jnp.maximum(acc, h)
    out_ref[...] = _lrelu(acc)


def _conv_max(x, xg, w, O):
    C = x.shape[-1]
    xg = xg.reshape(K, P, C)
    return pl.pallas_call(
        _conv_max_body,
        grid=(P // R,),
        in_specs=[
            pl.BlockSpec((R, C), lambda t: (t, 0)),
            pl.BlockSpec((K, R, C), lambda t: (0, t, 0)),
            pl.BlockSpec((O, 2 * C), lambda t: (0, 0)),
        ],
        out_specs=pl.BlockSpec((R, O), lambda t: (t, 0)),
        out_shape=jax.ShapeDtypeStruct((P, O), jnp.float32),
    )(x, xg, w)


def _head_body(x1_ref, x2_ref, x3_ref, x4_ref, w1_ref, w2_ref, w3_ref,
               w4_ref, l1_ref, l2_ref, b2_ref, out_ref):
    dims = (((1,), (1,)), ((), ()))
    h = lax.dot_general(x1_ref[...], w1_ref[...], dims,
                        preferred_element_type=jnp.float32)
    h = h + lax.dot_general(x2_ref[...], w2_ref[...], dims,
                            preferred_element_type=jnp.float32)
    h = h + lax.dot_general(x3_ref[...], w3_ref[...], dims,
                            preferred_element_type=jnp.float32)
    h = h + lax.dot_general(x4_ref[...], w4_ref[...], dims,
                            preferred_element_type=jnp.float32)
    h = _lrelu(h)
    p1 = jnp.max(h, axis=0, keepdims=True)
    p2 = jnp.sum(h, axis=0, keepdims=True) * (1.0 / N)
    g = jnp.concatenate([p1, p2], axis=1)
    g = _lrelu(lax.dot_general(g, l1_ref[...], dims,
                               preferred_element_type=jnp.float32))
    g = _lrelu(lax.dot_general(g, l2_ref[...], dims,
                               preferred_element_type=jnp.float32)
               + b2_ref[...])
    out_ref[0] = g


def _head(x1, x2, x3, x4, w51, w52, w53, w54, L1, L2, b2):
    full = lambda shape: pl.BlockSpec(shape, lambda b: tuple(0 for _ in shape))
    return pl.pallas_call(
        _head_body,
        grid=(B,),
        in_specs=[
            pl.BlockSpec((N, 64), lambda b: (b, 0)),
            pl.BlockSpec((N, 64), lambda b: (b, 0)),
            pl.BlockSpec((N, 128), lambda b: (b, 0)),
            pl.BlockSpec((N, 256), lambda b: (b, 0)),
            full((512, 64)),
            full((512, 64)),
            full((512, 128)),
            full((512, 256)),
            full((512, 1024)),
            full((256, 512)),
            full((1, 256)),
        ],
        out_specs=pl.BlockSpec((1, 1, 256), lambda b: (b, 0, 0)),
        out_shape=jax.ShapeDtypeStruct((B, 1, 256), jnp.float32),
    )(x1, x2, x3, x4, w51, w52, w53, w54, L1, L2, b2).reshape(B, 256)


def _edge_block(xt, W, O, SR):
    C = xt.shape[-1]
    idx = _knn(xt)
    idxk = jnp.transpose(idx.reshape(P, K)).reshape(K * P)
    xf = xt.reshape(P, C)
    xg = _gather_rows(xf, idxk, C, SR)
    return _conv_max(xf, xg, W, O).reshape(B, N, O)


def kernel(x, W1, W2, W3, W4, W5, L1, L2, b2):
    xt = jnp.transpose(x, (0, 2, 1))
    xt = jnp.pad(xt, ((0, 0), (0, 0), (0, 13)))
    W1p = jnp.pad(W1.reshape(64, 2, 3),
                  ((0, 0), (0, 0), (0, 13))).reshape(64, 32)

    x1 = _edge_block(xt, W1p, 64, 1024)
    x2 = _edge_block(x1, W2, 64, 1024)
    x3 = _edge_block(x2, W3, 128, 512)
    x4 = _edge_block(x3, W4, 256, 512)

    return _head(x1.reshape(P, 64), x2.reshape(P, 64),
                 x3.reshape(P, 128), x4.reshape(P, 256),
                 W5[:, 0:64], W5[:, 64:128], W5[:, 128:256], W5[:, 256:512],
                 L1, L2, b2.reshape(1, 256))


# speedup vs baseline: 21.0360x; 1.0263x over previous
import functools

import jax
import jax.numpy as jnp
from jax import lax
from jax.experimental import pallas as pl
from jax.experimental.pallas import tpu as pltpu
from jax.experimental.pallas import tpu_sc as plsc

B = 16
N = 2048
P = B * N
K = 10
R = 512
NEG = -1e30


def _lrelu(v):
    return jnp.maximum(v, 0.2 * v)


def _knn_body(xr_ref, xt_ref, idx_ref, nc_ref):
    b = pl.program_id(0)
    r = pl.program_id(1)
    xr = xr_ref[0]
    xt = xt_ref[0]

    @pl.when(r == 0)
    def _():
        nc_ref[...] = jnp.sum(xt * xt, axis=1, keepdims=True).T

    g = lax.dot_general(xr, xt, (((1,), (1,)), ((), ())),
                        preferred_element_type=jnp.float32)
    nr = jnp.sum(xr * xr, axis=1, keepdims=True)
    pw = (2.0 * g - nr) - nc_ref[...]

    col = lax.broadcasted_iota(jnp.int32, (R, N), 1)
    row_ids = r * R + lax.broadcasted_iota(jnp.int32, (R, 1), 0)
    pw = jnp.where(col == row_ids, NEG, pw)

    lane = lax.broadcasted_iota(jnp.int32, (R, 128), 1)
    idx_acc = jnp.where(lane == 0, row_ids, 0)
    a = None
    for t in range(1, K):
        if a is not None:
            pw = jnp.where(col == a, NEG, pw)
        val = pw[:, 0:128]
        vidx = lane
        for c in range(1, N // 128):
            ch = pw[:, c * 128:(c + 1) * 128]
            gt = ch > val
            val = jnp.maximum(val, ch)
            vidx = jnp.where(gt, lane + c * 128, vidx)
        m = jnp.max(val, axis=1, keepdims=True)
        cand = jnp.where(val == m, vidx, N)
        a = jnp.min(cand, axis=1, keepdims=True)
        idx_acc = jnp.where(lane == t, a, idx_acc)

    idx_ref[...] = jnp.transpose(idx_acc[:, :16])[:K] + b * N


def _knn(xt):
    C = xt.shape[-1]
    return pl.pallas_call(
        _knn_body,
        grid=(B, N // R),
        in_specs=[
            pl.BlockSpec((1, R, C), lambda b, r: (b, r, 0)),
            pl.BlockSpec((1, N, C), lambda b, r: (b, 0, 0)),
        ],
        out_specs=pl.BlockSpec((K, R), lambda b, r: (0, b * (N // R) + r)),
        out_shape=jax.ShapeDtypeStruct((K, P), jnp.int32),
        scratch_shapes=[pltpu.VMEM((1, N), jnp.float32)],
    )(xt, xt)


def _gather_rows(x, idxk, C, SR):
    TOT = K * P
    NW = 32
    chunk = TOT // NW
    nsub = chunk // SR
    mesh = plsc.VectorSubcoreMesh(core_axis_name="c", subcore_axis_name="s")

    @functools.partial(
        pl.kernel,
        out_type=jax.ShapeDtypeStruct((TOT, C), jnp.float32),
        mesh=mesh,
        scratch_types=[
            pltpu.VMEM((SR,), jnp.int32),
            pltpu.VMEM((SR,), jnp.int32),
            pltpu.VMEM((SR, C), jnp.float32),
            pltpu.VMEM((SR, C), jnp.float32),
            pltpu.SemaphoreType.DMA,
            pltpu.SemaphoreType.DMA,
            pltpu.SemaphoreType.DMA,
            pltpu.SemaphoreType.DMA,
        ],
        compiler_params=pltpu.CompilerParams(use_tc_tiling_on_sc=False),
    )
    def sc_kernel(x_hbm, idx_hbm, out_hbm, idx0, idx1, rows0, rows1,
                  g0, g1, s0, s1):
        wid = lax.axis_index("s") * 2 + lax.axis_index("c")
        base = wid * chunk
        bufs = ((idx0, rows0, g0, s0), (idx1, rows1, g1, s1))

        for p in range(2):
            idx_v, rows_v, gsem, _ = bufs[p]
            pltpu.sync_copy(idx_hbm.at[pl.ds(base + p * SR, SR)], idx_v)
            pltpu.async_copy(x_hbm.at[idx_v], rows_v, gsem)

        def pair(i, carry):
            i0 = i * 2
            for p in range(2):
                idx_v, rows_v, gsem, ssem = bufs[p]
                j = i0 + p
                r0 = base + j * SR
                pltpu.make_async_copy(x_hbm.at[idx_v], rows_v, gsem).wait()
                pltpu.async_copy(rows_v, out_hbm.at[pl.ds(r0, SR)], ssem)

                @pl.when(j + 2 < nsub)
                def _():
                    r2 = base + (j + 2) * SR
                    pltpu.sync_copy(idx_hbm.at[pl.ds(r2, SR)], idx_v)
                    pltpu.make_async_copy(
                        rows_v, out_hbm.at[pl.ds(r0, SR)], ssem).wait()
                    pltpu.async_copy(x_hbm.at[idx_v], rows_v, gsem)
            return carry

        lax.fori_loop(0, nsub // 2, pair, 0)
        for p in range(2):
            idx_v, rows_v, _, ssem = bufs[p]
            r0 = base + (nsub - 2 + p) * SR
            pltpu.make_async_copy(rows_v, out_hbm.at[pl.ds(r0, SR)],
                                  ssem).wait()

    return sc_kernel(x, idxk)


def _conv_max_body(x_ref, xg_ref, w_ref, out_ref):
    xi = x_ref[...]
    w = w_ref[...]
    acc = None
    for j in range(K):
        xj = xg_ref[j]
        f2 = jnp.concatenate([xj - xi, xi], axis=1)
        h = lax.dot_general(f2, w, (((1,), (1,)), ((), ())),
                            preferred_element_type=jnp.float32)
        acc = h if acc is None else jnp.maximum(acc, h)
    out_ref[...] = _lrelu(acc)


def _conv_max(x, xg, w, O):
    C = x.shape[-1]
    xg = xg.reshape(K, P, C)
    return pl.pallas_call(
        _conv_max_body,
        grid=(P // R,),
        in_specs=[
            pl.BlockSpec((R, C), lambda t: (t, 0)),
            pl.BlockSpec((K, R, C), lambda t: (0, t, 0)),
            pl.BlockSpec((O, 2 * C), lambda t: (0, 0)),
        ],
        out_specs=pl.BlockSpec((R, O), lambda t: (t, 0)),
        out_shape=jax.ShapeDtypeStruct((P, O), jnp.float32),
    )(x, xg, w)


def _head_body(x1_ref, x2_ref, x3_ref, x4_ref, w1_ref, w2_ref, w3_ref,
               w4_ref, l1_ref, l2_ref, b2_ref, out_ref):
    dims = (((1,), (1,)), ((), ()))
    h = lax.dot_general(x1_ref[...], w1_ref[...], dims,
                        preferred_element_type=jnp.float32)
    h = h + lax.dot_general(x2_ref[...], w2_ref[...], dims,
                            preferred_element_type=jnp.float32)
    h = h + lax.dot_general(x3_ref[...], w3_ref[...], dims,
                            preferred_element_type=jnp.float32)
    h = h + lax.dot_general(x4_ref[...], w4_ref[...], dims,
                            preferred_element_type=jnp.float32)
    h = _lrelu(h)
    p1 = jnp.max(h, axis=0, keepdims=True)
    p2 = jnp.sum(h, axis=0, keepdims=True) * (1.0 / N)
    g = jnp.concatenate([p1, p2], axis=1)
    g = _lrelu(lax.dot_general(g, l1_ref[...], dims,
                               preferred_element_type=jnp.float32))
    g = _lrelu(lax.dot_general(g, l2_ref[...], dims,
                               preferred_element_type=jnp.float32)
               + b2_ref[...])
    out_ref[0] = g


def _head(x1, x2, x3, x4, w51, w52, w53, w54, L1, L2, b2):
    full = lambda shape: pl.BlockSpec(shape, lambda b: tuple(0 for _ in shape))
    return pl.pallas_call(
        _head_body,
        grid=(B,),
        in_specs=[
            pl.BlockSpec((N, 64), lambda b: (b, 0)),
            pl.BlockSpec((N, 64), lambda b: (b, 0)),
            pl.BlockSpec((N, 128), lambda b: (b, 0)),
            pl.BlockSpec((N, 256), lambda b: (b, 0)),
            full((512, 64)),
            full((512, 64)),
            full((512, 128)),
            full((512, 256)),
            full((512, 1024)),
            full((256, 512)),
            full((1, 256)),
        ],
        out_specs=pl.BlockSpec((1, 1, 256), lambda b: (b, 0, 0)),
        out_shape=jax.ShapeDtypeStruct((B, 1, 256), jnp.float32),
    )(x1, x2, x3, x4, w51, w52, w53, w54, L1, L2, b2).reshape(B, 256)


def _edge_block(xt, W, O, SR):
    C = xt.shape[-1]
    idxk = _knn(xt).reshape(K * P)
    xf = xt.reshape(P, C)
    xg = _gather_rows(xf, idxk, C, SR)
    return _conv_max(xf, xg, W, O).reshape(B, N, O)


def kernel(x, W1, W2, W3, W4, W5, L1, L2, b2):
    xt = jnp.transpose(x, (0, 2, 1))
    xt = jnp.pad(xt, ((0, 0), (0, 0), (0, 13)))
    W1p = jnp.pad(W1.reshape(64, 2, 3),
                  ((0, 0), (0, 0), (0, 13))).reshape(64, 32)

    x1 = _edge_block(xt, W1p, 64, 1024)
    x2 = _edge_block(x1, W2, 64, 512)
    x3 = _edge_block(x2, W3, 128, 512)
    x4 = _edge_block(x3, W4, 256, 256)

    return _head(x1.reshape(P, 64), x2.reshape(P, 64),
                 x3.reshape(P, 128), x4.reshape(P, 256),
                 W5[:, 0:64], W5[:, 64:128], W5[:, 128:256], W5[:, 256:512],
                 L1, L2, b2.reshape(1, 256))
